# Initial kernel scaffold; baseline (speedup 1.0000x reference)
import jax
import jax.numpy as jnp
from jax.experimental import pallas as pl


def kernel(x, levels):
    raise NotImplementedError("write your pallas kernel here")


# speedup vs baseline: 1.0428x; 1.0428x over previous
import jax
import jax.numpy as jnp
from jax.experimental import pallas as pl
from jax.experimental.pallas import tpu as pltpu


def _tc_body(levels_ref, x_ref, o_ref):
    l0 = levels_ref[0]
    l1 = levels_ref[1]
    thr = (l0 + l1) * 0.5
    o_ref[...] = jnp.where(x_ref[...] > thr, l1, l0)


def kernel(x, levels):
    M, N = x.shape
    BM = 256
    return pl.pallas_call(
        _tc_body,
        grid=(M // BM,),
        in_specs=[
            pl.BlockSpec(memory_space=pltpu.SMEM),
            pl.BlockSpec((BM, N), lambda i: (i, 0)),
        ],
        out_specs=pl.BlockSpec((BM, N), lambda i: (i, 0)),
        out_shape=jax.ShapeDtypeStruct((M, N), x.dtype),
    )(levels, x)
